# Initial kernel scaffold; baseline (speedup 1.0000x reference)
import jax
import jax.numpy as jnp
from jax.experimental import pallas as pl


def kernel(edge_index, edges, emb, W1, b1, W2, b2, Wp1, bp1, Wp2, bp2):
    raise NotImplementedError("write your pallas kernel here")


# speedup vs baseline: 10.0552x; 10.0552x over previous
import functools

import jax
import jax.numpy as jnp
from jax import lax
from jax.experimental import pallas as pl
from jax.experimental.pallas import tpu as pltpu
from jax.experimental.pallas import tpu_sc as plsc

N = 10000
E = 320000
Q = 65536
D = 128
NP = 10240

NC = 2
NS = 16
NW = NC * NS

ECHUNK = 80
ENCHUNK = (E // NW) // ECHUNK
EPT = E // NW

QPT = Q // NW
QCHUNK = 128
QNCHUNK = QPT // QCHUNK

ROWS_PER_TILE = NP // NS

_MESH = plsc.VectorSubcoreMesh(core_axis_name="c", subcore_axis_name="s")


DEGW = 128


@functools.partial(
    pl.kernel,
    out_type=jax.ShapeDtypeStruct((NC * NP, DEGW), jnp.float32),
    mesh=_MESH,
    scratch_types=[
        pltpu.VMEM((ECHUNK,), jnp.int32),
        pltpu.VMEM((ECHUNK, DEGW), jnp.float32),
        pltpu.VMEM_SHARED((NP, DEGW), jnp.float32),
    ],
)
def _deg_sc(dst_hbm, ones_hbm, zeros_hbm, out_hbm, dbuf, onesv, deg):
    cid = lax.axis_index("c")
    sid = lax.axis_index("s")
    wid = cid * NS + sid
    r0 = sid * ROWS_PER_TILE
    pltpu.sync_copy(zeros_hbm, deg.at[pl.ds(r0, ROWS_PER_TILE)])
    pltpu.sync_copy(ones_hbm, onesv)
    plsc.subcore_barrier()

    def body(j, carry):
        pltpu.sync_copy(dst_hbm.at[wid, j], dbuf)
        pltpu.sync_copy(onesv, deg.at[dbuf], add=True)
        return carry

    lax.fori_loop(0, ENCHUNK, body, 0)
    plsc.subcore_barrier()
    pltpu.sync_copy(deg.at[pl.ds(r0, ROWS_PER_TILE)],
                    out_hbm.at[pl.ds(cid * NP + r0, ROWS_PER_TILE)])


@functools.partial(
    pl.kernel,
    out_type=jax.ShapeDtypeStruct((NC * NP, D), jnp.float32),
    mesh=_MESH,
    scratch_types=[
        pltpu.VMEM((ECHUNK,), jnp.int32),
        pltpu.VMEM((ECHUNK,), jnp.int32),
        pltpu.VMEM((ECHUNK, D), jnp.float32),
        pltpu.VMEM_SHARED((NP, D), jnp.float32),
        pltpu.SemaphoreType.DMA,
    ],
)
def _conv_sc(h_hbm, src_hbm, dst_hbm, out_hbm, sbuf, dbuf, rows, acc, sem):
    cid = lax.axis_index("c")
    sid = lax.axis_index("s")
    wid = cid * NS + sid
    r0 = sid * ROWS_PER_TILE
    pltpu.sync_copy(h_hbm.at[pl.ds(r0, ROWS_PER_TILE)],
                    acc.at[pl.ds(r0, ROWS_PER_TILE)])
    plsc.subcore_barrier()

    def body(j, carry):
        pltpu.sync_copy(src_hbm.at[wid, j], sbuf)
        pltpu.sync_copy(dst_hbm.at[wid, j], dbuf)
        pltpu.async_copy(h_hbm.at[sbuf], rows, sem).wait()
        pltpu.sync_copy(rows, acc.at[dbuf], add=True)
        return carry

    lax.fori_loop(0, ENCHUNK, body, 0)
    plsc.subcore_barrier()
    pltpu.sync_copy(acc.at[pl.ds(r0, ROWS_PER_TILE)],
                    out_hbm.at[pl.ds(cid * NP + r0, ROWS_PER_TILE)])


@functools.partial(
    pl.kernel,
    out_type=[
        jax.ShapeDtypeStruct((Q, D), jnp.float32),
        jax.ShapeDtypeStruct((Q, D), jnp.float32),
    ],
    mesh=_MESH,
    scratch_types=[
        pltpu.VMEM((QCHUNK,), jnp.int32),
        pltpu.VMEM((QCHUNK, D), jnp.float32),
        pltpu.SemaphoreType.DMA,
    ],
)
def _link_sc(x_hbm, q0_hbm, q1_hbm, ga_hbm, gb_hbm, ibuf, rows, sem):
    cid = lax.axis_index("c")
    sid = lax.axis_index("s")
    wid = cid * NS + sid

    def body(j, carry):
        base = wid * QPT + j * QCHUNK
        pltpu.sync_copy(q0_hbm.at[wid, j], ibuf)
        pltpu.async_copy(x_hbm.at[ibuf], rows, sem).wait()
        pltpu.sync_copy(rows, ga_hbm.at[pl.ds(base, QCHUNK)])
        pltpu.sync_copy(q1_hbm.at[wid, j], ibuf)
        pltpu.async_copy(x_hbm.at[ibuf], rows, sem).wait()
        pltpu.sync_copy(rows, gb_hbm.at[pl.ds(base, QCHUNK)])
        return carry

    lax.fori_loop(0, QNCHUNK, body, 0)


_RB = 1024


def _dis_body(degp_ref, dis_ref):
    d = degp_ref[...]
    deg = d[0:NP, 0:1] + d[NP:2 * NP, 0:1] + 1.0
    dis_ref[...] = lax.rsqrt(deg)


def _dis_tc(degp):
    return pl.pallas_call(
        _dis_body,
        grid=(1,),
        in_specs=[pl.BlockSpec((NC * NP, DEGW), lambda i: (0, 0))],
        out_specs=pl.BlockSpec((NP, 1), lambda i: (0, 0)),
        out_shape=jax.ShapeDtypeStruct((NP, 1), jnp.float32),
    )(degp)


def _prep_body(dis_ref, emb_ref, w1_ref, h_ref):
    h_ref[...] = jnp.dot(emb_ref[...] * dis_ref[...], w1_ref[...],
                         preferred_element_type=jnp.float32)


def _prep_tc(dis, emb, w1):
    return pl.pallas_call(
        _prep_body,
        grid=(NP // _RB,),
        in_specs=[
            pl.BlockSpec((_RB, 1), lambda i: (i, 0)),
            pl.BlockSpec((_RB, D), lambda i: (i, 0)),
            pl.BlockSpec((D, D), lambda i: (0, 0)),
        ],
        out_specs=pl.BlockSpec((_RB, D), lambda i: (i, 0)),
        out_shape=jax.ShapeDtypeStruct((NP, D), jnp.float32),
    )(dis, emb, w1)


def _mid_body(acca_ref, accb_ref, hp_ref, dis_ref, b_ref, w_ref, out_ref):
    s = acca_ref[...] + accb_ref[...] - hp_ref[...]
    x1 = jnp.maximum(dis_ref[...] * s + b_ref[...], 0.0)
    out_ref[...] = jnp.dot(x1 * dis_ref[...], w_ref[...],
                           preferred_element_type=jnp.float32)


def _mid_tc(acc, hp, dis, b_row, w2):
    return pl.pallas_call(
        _mid_body,
        grid=(NP // _RB,),
        in_specs=[
            pl.BlockSpec((_RB, D), lambda i: (i, 0)),
            pl.BlockSpec((_RB, D), lambda i: (i + NP // _RB, 0)),
            pl.BlockSpec((_RB, D), lambda i: (i, 0)),
            pl.BlockSpec((_RB, 1), lambda i: (i, 0)),
            pl.BlockSpec((1, D), lambda i: (0, 0)),
            pl.BlockSpec((D, D), lambda i: (0, 0)),
        ],
        out_specs=pl.BlockSpec((_RB, D), lambda i: (i, 0)),
        out_shape=jax.ShapeDtypeStruct((NP, D), jnp.float32),
    )(acc, acc, hp, dis, b_row, w2)


def _final_body(acca_ref, accb_ref, hp_ref, dis_ref, b_ref, out_ref):
    s = acca_ref[...] + accb_ref[...] - hp_ref[...]
    out_ref[...] = dis_ref[...] * s + b_ref[...]


def _final_tc(acc, hp, dis, b_row):
    return pl.pallas_call(
        _final_body,
        grid=(NP // _RB,),
        in_specs=[
            pl.BlockSpec((_RB, D), lambda i: (i, 0)),
            pl.BlockSpec((_RB, D), lambda i: (i + NP // _RB, 0)),
            pl.BlockSpec((_RB, D), lambda i: (i, 0)),
            pl.BlockSpec((_RB, 1), lambda i: (i, 0)),
            pl.BlockSpec((1, D), lambda i: (0, 0)),
        ],
        out_specs=pl.BlockSpec((_RB, D), lambda i: (i, 0)),
        out_shape=jax.ShapeDtypeStruct((NP, D), jnp.float32),
    )(acc, acc, hp, dis, b_row)


_QB = 2048


def _pred_body(ga_ref, gb_ref, wp1_ref, bp1_ref, wp2_ref, bp2_ref, out_ref):
    h = ga_ref[...] * gb_ref[...]
    h = jnp.maximum(
        jnp.dot(h, wp1_ref[...], preferred_element_type=jnp.float32)
        + bp1_ref[...], 0.0)
    z = jnp.dot(h, wp2_ref[...], preferred_element_type=jnp.float32) \
        + bp2_ref[...]
    out_ref[...] = jax.nn.sigmoid(z)


def _pred_tc(ga, gb, wp1, bp1_row, wp2, bp2_row):
    return pl.pallas_call(
        _pred_body,
        grid=(Q // _QB,),
        in_specs=[
            pl.BlockSpec((_QB, D), lambda i: (i, 0)),
            pl.BlockSpec((_QB, D), lambda i: (i, 0)),
            pl.BlockSpec((D, D), lambda i: (0, 0)),
            pl.BlockSpec((1, D), lambda i: (0, 0)),
            pl.BlockSpec((D, 1), lambda i: (0, 0)),
            pl.BlockSpec((1, 1), lambda i: (0, 0)),
        ],
        out_specs=pl.BlockSpec((_QB, 1), lambda i: (i, 0)),
        out_shape=jax.ShapeDtypeStruct((Q, 1), jnp.float32),
    )(ga, gb, wp1, bp1_row, wp2, bp2_row)


def kernel(edge_index, edges, emb, W1, b1, W2, b2, Wp1, bp1, Wp2, bp2):
    src = edge_index[0].astype(jnp.int32)
    dst = edge_index[1].astype(jnp.int32)
    src3 = src.reshape(NW, ENCHUNK, ECHUNK)
    dst3 = dst.reshape(NW, ENCHUNK, ECHUNK)
    q0 = edges[0].astype(jnp.int32).reshape(NW, QNCHUNK, QCHUNK)
    q1 = edges[1].astype(jnp.int32).reshape(NW, QNCHUNK, QCHUNK)

    emb_p = jnp.pad(emb, ((0, NP - N), (0, 0)))
    degp = _deg_sc(dst3,
                   jnp.ones((ECHUNK, DEGW), jnp.float32),
                   jnp.zeros((ROWS_PER_TILE, DEGW), jnp.float32))
    dis = _dis_tc(degp)
    h1p = _prep_tc(dis, emb_p, W1)
    acc1 = _conv_sc(h1p, src3, dst3)
    h2p = _mid_tc(acc1, h1p, dis, b1.reshape(1, D), W2)
    acc2 = _conv_sc(h2p, src3, dst3)
    x2 = _final_tc(acc2, h2p, dis, b2.reshape(1, D))
    ga, gb = _link_sc(x2, q0, q1)
    out = _pred_tc(ga, gb, Wp1, bp1.reshape(1, D), Wp2, bp2.reshape(1, 1))
    return out[:, 0]
